# baseline (speedup 1.0000x reference)
import functools
import jax
import jax.numpy as jnp
from jax import lax
from jax.experimental import pallas as pl
from jax.experimental.pallas import tpu as pltpu
from jax.experimental.pallas import tpu_sc as plsc

NE = 1024
ED = 256
PTH = 0.8
B, T = 16, 2048
NC, NS, L = 2, 16, 16
NW = NC * NS
HALF = T // 2
CH = 64
WLEN = CH + 16
NEP = 1032
NCHUNK = HALF // CH
GP = CH // L


def _sc_body(p_hbm, w_hbm, out_hbm, part_hbm,
             p_row, o1b, o2b, pfb, winb, outb, accb, sw0, sw1, so0, so1):
  sem_w = (sw0, sw1)
  sem_o = (so0, so1)
  c = lax.axis_index("c")
  s = lax.axis_index("s")
  wid = s * NC + c
  b = wid // 2
  half = wid % 2
  t0 = half * HALF
  row_base = b * T + t0

  pltpu.sync_copy(p_hbm.at[b], p_row)

  iota = lax.iota(jnp.int32, L)

  def _carry_body(i, acc):
    pv = p_row[pl.ds(i * L, L)]
    pos = i * L + iota
    sig = (pv >= PTH) & (pos > 0)
    return acc + jnp.where(sig, 1, 0).astype(jnp.int32)

  carry_vec = lax.fori_loop(0, half * (HALF // L), _carry_body,
                            jnp.zeros((L,), jnp.int32))
  cum = jnp.sum(carry_vec)

  win_cp = [None, None]
  out_cp = [None, None]

  def build_idx(ch, cum):
    buf = ch % 2
    lo = jnp.minimum((jnp.maximum(cum - 1, 0) // 8) * 8, NEP - WLEN)
    base = t0 + ch * CH
    for j in range(GP):
      pv = p_row[pl.ds(base + j * L, L)]
      pos = base + j * L + iota
      sig = (pv >= PTH) & (pos > 0)
      sigi = jnp.where(sig, 1, 0).astype(jnp.int32)
      loc = plsc.cumsum(sigi) + cum
      i1 = jnp.minimum(loc, NE - 1)
      i2 = jnp.clip(jnp.where(sig, i1 - 1, i1 + 1), 0, NE)
      pf = jnp.where(sig, pv, 1.0 - pv)
      o1b[buf, pl.ds(j * L, L)] = i1 - lo
      o2b[buf, pl.ds(j * L, L)] = i2 - lo
      pfb[buf, pl.ds(j * L, L)] = pf
      cum = jnp.max(loc)
    return cum, lo

  def blend(ch, acc_s, acc_q):
    buf = ch % 2
    win_cp[buf].wait()
    if out_cp[buf] is not None:
      out_cp[buf].wait()
    wref = winb.at[buf]
    oref = outb.at[buf]

    def group_body(g, carry):
      o1 = o1b[buf, pl.ds(g * L, L)]
      o2 = o2b[buf, pl.ds(g * L, L)]
      pf = pfb[buf, pl.ds(g * L, L)]
      riota = g * L + iota

      def col_body(cc, carry2):
        a_s, a_q = carry2
        for k in range(L):
          col = cc * L + k
          csplat = jnp.full((L,), col, jnp.int32)
          z1 = plsc.load_gather(wref, [o1, csplat])
          z2 = plsc.load_gather(wref, [o2, csplat])
          plsc.store_scatter(oref, [riota, csplat], z2 + pf * (z1 - z2))
          a_s = a_s + z1
          a_q = a_q + z1 * z1
        return (a_s, a_q)

      return lax.fori_loop(0, ED // L, col_body, carry)

    acc_s, acc_q = lax.fori_loop(0, GP, group_body, (acc_s, acc_q))
    out_cp[buf] = pltpu.async_copy(
        oref, out_hbm.at[pl.ds(row_base + ch * CH, CH)], sem_o[buf])
    return acc_s, acc_q

  acc_s = jnp.zeros((L,), jnp.float32)
  acc_q = jnp.zeros((L,), jnp.float32)

  for ch in range(NCHUNK):
    buf = ch % 2
    cum, lo = build_idx(ch, cum)
    win_cp[buf] = pltpu.async_copy(
        w_hbm.at[pl.ds(lo, WLEN)], winb.at[buf], sem_w[buf])
    if ch > 0:
      acc_s, acc_q = blend(ch - 1, acc_s, acc_q)
  acc_s, acc_q = blend(NCHUNK - 1, acc_s, acc_q)
  out_cp[0].wait()
  out_cp[1].wait()

  accb[pl.ds(0, L)] = acc_s
  accb[pl.ds(L, L)] = acc_q
  pltpu.sync_copy(accb, part_hbm.at[wid])


_vq3_sc = functools.partial(
    pl.kernel,
    out_type=(jax.ShapeDtypeStruct((B * T, ED), jnp.float32),
              jax.ShapeDtypeStruct((NW, 2 * L), jnp.float32)),
    mesh=plsc.VectorSubcoreMesh(core_axis_name="c", subcore_axis_name="s",
                                num_cores=NC, num_subcores=NS),
    compiler_params=pltpu.CompilerParams(needs_layout_passes=False),
    scratch_types=[
        pltpu.VMEM((T,), jnp.float32),
        pltpu.VMEM((2, CH), jnp.int32),
        pltpu.VMEM((2, CH), jnp.int32),
        pltpu.VMEM((2, CH), jnp.float32),
        pltpu.VMEM((2, WLEN, ED), jnp.float32),
        pltpu.VMEM((2, CH, ED), jnp.float32),
        pltpu.VMEM((2 * L,), jnp.float32),
        pltpu.SemaphoreType.DMA,
        pltpu.SemaphoreType.DMA,
        pltpu.SemaphoreType.DMA,
        pltpu.SemaphoreType.DMA,
    ],
)(_sc_body)


def kernel(p_change, weight):
  w_pad = jnp.zeros((NEP, ED), jnp.float32).at[:NE + 1].set(weight)
  z_flat, parts = _vq3_sc(p_change, w_pad)
  z_out = z_flat.reshape(B, T, ED)
  n = B * T * ED
  ssum = jnp.sum(parts[:, :L])
  qsum = jnp.sum(parts[:, L:])
  v = (qsum - ssum * ssum / n) / (n - 1)
  return (z_out, v)


# speedup vs baseline: 3.8935x; 3.8935x over previous
import functools
import jax
import jax.numpy as jnp
from jax import lax
from jax.experimental import pallas as pl
from jax.experimental.pallas import tpu as pltpu
from jax.experimental.pallas import tpu_sc as plsc

NE = 1024
ED = 256
PTH = 0.8
B, T = 16, 2048
NC, NS, L = 2, 16, 16
NW = NC * NS
HALF = T // 2
CH = 64
NCHUNK = HALF // CH
GP = CH // L
CPR = ED // L


def _sc_body(p_hbm, w_hbm, out_hbm, part_hbm,
             p_row,
             idx1_0, idx1_1, idx2_0, idx2_1, pfr_0, pfr_1,
             z1_0, z1_1, z2_0, z2_1, ob_0, ob_1, accb,
             sg1_0, sg1_1, sg2_0, sg2_1, so_0, so_1):
  idx1 = (idx1_0, idx1_1)
  idx2 = (idx2_0, idx2_1)
  pfr = (pfr_0, pfr_1)
  z1b = (z1_0, z1_1)
  z2b = (z2_0, z2_1)
  outb = (ob_0, ob_1)
  sg1 = (sg1_0, sg1_1)
  sg2 = (sg2_0, sg2_1)
  so = (so_0, so_1)

  c = lax.axis_index("c")
  s = lax.axis_index("s")
  wid = s * NC + c
  b = wid // 2
  half = wid % 2
  t0 = half * HALF
  row_base = b * T + t0

  pltpu.sync_copy(p_hbm.at[b], p_row)

  iota = lax.iota(jnp.int32, L)

  def _carry_body(i, acc):
    pv = p_row[pl.ds(i * L, L)]
    pos = i * L + iota
    sig = (pv >= PTH) & (pos > 0)
    return acc + jnp.where(sig, 1, 0).astype(jnp.int32)

  carry_vec = lax.fori_loop(0, half * (HALF // L), _carry_body,
                            jnp.zeros((L,), jnp.int32))
  cum = jnp.sum(carry_vec)

  g1_cp = [None, None]
  g2_cp = [None, None]
  out_cp = [None, None]

  def build_idx(ch, cum):
    buf = ch % 2
    base = t0 + ch * CH
    for j in range(GP):
      pv = p_row[pl.ds(base + j * L, L)]
      pos = base + j * L + iota
      sig = (pv >= PTH) & (pos > 0)
      sigi = jnp.where(sig, 1, 0).astype(jnp.int32)
      loc = plsc.cumsum(sigi) + cum
      i1 = jnp.minimum(loc, NE - 1)
      i2 = jnp.clip(jnp.where(sig, i1 - 1, i1 + 1), 0, NE)
      pf = jnp.where(sig, pv, 1.0 - pv)
      idx1[buf][pl.ds(j * L, L)] = i1
      idx2[buf][pl.ds(j * L, L)] = i2
      scat_base = j * (L * L) + iota * L
      for k in range(L):
        plsc.store_scatter(pfr[buf], [scat_base + k], pf)
      cum = jnp.max(loc)
    return cum

  def blend(ch, acc_s, acc_q):
    buf = ch % 2
    g1_cp[buf].wait()
    g2_cp[buf].wait()

    def _blend_body(r, bl_carry):
      a_s, a_q = bl_carry
      pf = pfr[buf][pl.ds(r * L, L)]
      for cix in range(CPR):
        z1 = z1b[buf][r, pl.ds(cix * L, L)]
        z2 = z2b[buf][r, pl.ds(cix * L, L)]
        outb[buf][r, pl.ds(cix * L, L)] = z2 + pf * (z1 - z2)
        a_s = a_s + z1
        a_q = a_q + z1 * z1
      return (a_s, a_q)

    acc_s, acc_q = lax.fori_loop(0, CH, _blend_body, (acc_s, acc_q))
    out_cp[buf] = pltpu.async_copy(
        outb[buf], out_hbm.at[pl.ds(row_base + ch * CH, CH)], so[buf])
    return acc_s, acc_q

  acc_s = jnp.zeros((L,), jnp.float32)
  acc_q = jnp.zeros((L,), jnp.float32)

  for ch in range(NCHUNK):
    buf = ch % 2
    cum = build_idx(ch, cum)
    if out_cp[buf] is not None:
      out_cp[buf].wait()
    g1_cp[buf] = pltpu.async_copy(w_hbm.at[idx1[buf]], z1b[buf], sg1[buf])
    g2_cp[buf] = pltpu.async_copy(w_hbm.at[idx2[buf]], z2b[buf], sg2[buf])
    if ch > 0:
      acc_s, acc_q = blend(ch - 1, acc_s, acc_q)
  acc_s, acc_q = blend(NCHUNK - 1, acc_s, acc_q)
  out_cp[0].wait()
  out_cp[1].wait()

  accb[pl.ds(0, L)] = acc_s
  accb[pl.ds(L, L)] = acc_q
  pltpu.sync_copy(accb, part_hbm.at[wid])


_vq3_sc = functools.partial(
    pl.kernel,
    out_type=(jax.ShapeDtypeStruct((B * T, ED), jnp.float32),
              jax.ShapeDtypeStruct((NW, 2 * L), jnp.float32)),
    mesh=plsc.VectorSubcoreMesh(core_axis_name="c", subcore_axis_name="s",
                                num_cores=NC, num_subcores=NS),
    compiler_params=pltpu.CompilerParams(needs_layout_passes=False),
    scratch_types=[
        pltpu.VMEM((T,), jnp.float32),
        pltpu.VMEM((CH,), jnp.int32),
        pltpu.VMEM((CH,), jnp.int32),
        pltpu.VMEM((CH,), jnp.int32),
        pltpu.VMEM((CH,), jnp.int32),
        pltpu.VMEM((CH * L,), jnp.float32),
        pltpu.VMEM((CH * L,), jnp.float32),
        pltpu.VMEM((CH, ED), jnp.float32),
        pltpu.VMEM((CH, ED), jnp.float32),
        pltpu.VMEM((CH, ED), jnp.float32),
        pltpu.VMEM((CH, ED), jnp.float32),
        pltpu.VMEM((CH, ED), jnp.float32),
        pltpu.VMEM((CH, ED), jnp.float32),
        pltpu.VMEM((2 * L,), jnp.float32),
        pltpu.SemaphoreType.DMA,
        pltpu.SemaphoreType.DMA,
        pltpu.SemaphoreType.DMA,
        pltpu.SemaphoreType.DMA,
        pltpu.SemaphoreType.DMA,
        pltpu.SemaphoreType.DMA,
    ],
)(_sc_body)


def kernel(p_change, weight):
  z_flat, parts = _vq3_sc(p_change, weight)
  z_out = z_flat.reshape(B, T, ED)
  n = B * T * ED
  ssum = jnp.sum(parts[:, :L])
  qsum = jnp.sum(parts[:, L:])
  v = (qsum - ssum * ssum / n) / (n - 1)
  return (z_out, v)
